# baseline (speedup 1.0000x reference)
import functools

import jax
import jax.numpy as jnp
from jax import lax
from jax.experimental import pallas as pl
from jax.experimental.pallas import tpu as pltpu
from jax.experimental.pallas import tpu_sc as plsc

N = 10000
NC = 2
NS = 16
NW = NC * NS
L = 16
R = 320
EB = 128
SB = 1024
QCAP = 1184
BR = 256

_mesh = functools.partial(
    plsc.VectorSubcoreMesh, core_axis_name="c", subcore_axis_name="s")


def _zero_rows(ref, nrows, ncols):
    z = jnp.zeros((L,), jnp.float32)
    fw = ncols // L

    def body(t, carry):
        i = t // fw
        j = t - i * fw
        ref[i, pl.ds(j * L, L)] = z
        return carry

    lax.fori_loop(0, nrows * fw, body, 0)


def _zero_queue(qref):
    z = jnp.zeros((L,), jnp.int32)

    def body(i, carry):
        qref[pl.ds(i * L, L)] = z
        return carry

    lax.fori_loop(0, QCAP // L, body, 0)


def _scan_block(srcb, dstb, qpk, base, lo, rr, qn, drain, with_src):
    offs = []
    for ch in range(SB // L):
        off = base + ch * L
        d16 = dstb[pl.ds(off, L)]
        ok = (d16 - lo).astype(jnp.uint32) < rr
        offs.append(qn)
        qn = qn + plsc.all_reduce_population_count(ok)[0]
    for ch in range(SB // L):
        off = base + ch * L
        d16 = dstb[pl.ds(off, L)]
        loc = d16 - lo
        ok = (loc >= 0) & (loc < rr)
        if with_src:
            s16 = srcb[pl.ds(off, L)]
            val = s16 * 1024 + loc
        else:
            val = loc
        key = jnp.where(ok, 0, 1)
        _, vs = plsc.sort_key_val(key, val)
        qpk[pl.ds(offs[ch], L)] = vs
    return lax.while_loop(lambda q: q >= EB, drain, qn)


def _shift_queue(qref):
    for k in range((QCAP - EB) // L):
        t = qref[pl.ds(EB + k * L, L)]
        qref[pl.ds(k * L, L)] = t


def _make_agg(e_pad, F, mode):
    edge_split = mode == "edge"
    nblk = e_pad // SB // 2 if edge_split else e_pad // SB
    rr = 632
    fw = F // L
    out_shape = (2, N, F) if edge_split else (N, 2 * F)

    @functools.partial(
        pl.kernel,
        out_type=jax.ShapeDtypeStruct(out_shape, jnp.float32),
        mesh=_mesh(),
        compiler_params=pltpu.CompilerParams(needs_layout_passes=False),
        scratch_types=[
            pltpu.VMEM((2 * SB,), jnp.int32),
            pltpu.VMEM((2 * SB,), jnp.int32),
            pltpu.VMEM((QCAP,), jnp.int32),
            pltpu.VMEM((EB,), jnp.int32),
            pltpu.VMEM((EB, F), jnp.float32),
            pltpu.VMEM((rr, F), jnp.float32),
            pltpu.SemaphoreType.DMA,
            pltpu.SemaphoreType.DMA,
            pltpu.SemaphoreType.DMA,
        ],
    )
    def agg_kernel(g_hbm, src_hbm, dst_hbm, out_hbm,
                   srcb, dstb, qpk, qsrc, rows, acc, gsem, is0, is1):
        c = lax.axis_index("c")
        s = lax.axis_index("s")
        lo = s * rr
        kb = c * nblk if edge_split else 0
        gbase = 0 if edge_split else c * N

        _zero_rows(acc, rr, F)
        _zero_queue(qpk)

        def issue_block(k, p, sem):
            pltpu.async_copy(src_hbm.at[pl.ds((kb + k) * SB, SB)],
                             srcb.at[pl.ds(p * SB, SB)], sem)
            pltpu.async_copy(dst_hbm.at[pl.ds((kb + k) * SB, SB)],
                             dstb.at[pl.ds(p * SB, SB)], sem)

        def wait_block(p, sem):
            pltpu.make_async_copy(src_hbm.at[pl.ds(0, SB)],
                                  srcb.at[pl.ds(p * SB, SB)], sem).wait()
            pltpu.make_async_copy(dst_hbm.at[pl.ds(0, SB)],
                                  dstb.at[pl.ds(p * SB, SB)], sem).wait()

        def do_add(i, r):
            for j in range(fw):
                sl = pl.ds(j * L, L)
                acc[r, sl] = acc[r, sl] + rows[i, sl]

        def drain(qn):
            for k in range(EB // L):
                t = qpk[pl.ds(k * L, L)]
                qsrc[pl.ds(k * L, L)] = jnp.minimum(
                    lax.shift_right_logical(t, 10), N - 1) + gbase
            pltpu.async_copy(g_hbm.at[qsrc], rows, gsem).wait()

            def add16(i, carry):
                e = i * L
                tm = jnp.bitwise_and(qpk[pl.ds(e, L)], 1023)
                for u in range(L):
                    r = tm[u]
                    for j in range(fw):
                        sl = pl.ds(j * L, L)
                        acc[r, sl] = acc[r, sl] + rows[e + u, sl]
                return carry

            lax.fori_loop(0, EB // L, add16, 0)
            _shift_queue(qpk)
            return qn - EB

        issue_block(0, 0, is0)

        def body(k, qn):
            p = lax.rem(k, 2)

            @pl.when(p == 0)
            def _():
                wait_block(0, is0)

            @pl.when(p == 1)
            def _():
                wait_block(1, is1)

            @pl.when((p == 0) & (k + 1 < nblk))
            def _():
                issue_block(k + 1, 1, is1)

            @pl.when((p == 1) & (k + 1 < nblk))
            def _():
                issue_block(k + 1, 0, is0)

            return _scan_block(srcb, dstb, qpk, p * SB, lo, rr, qn,
                               drain, True)

        qn = lax.fori_loop(0, nblk, body, 0)

        for k in range(EB // L):
            t = qpk[pl.ds(k * L, L)]
            qsrc[pl.ds(k * L, L)] = jnp.minimum(
                lax.shift_right_logical(t, 10), N - 1) + gbase
        pltpu.async_copy(g_hbm.at[qsrc], rows, gsem).wait()

        def add_tail(i, carry):
            do_add(i, jnp.bitwise_and(qpk[pl.ds(i, L)][0], 1023))
            return carry

        lax.fori_loop(0, qn, add_tail, 0)

        tail_r = N - (NS - 1) * rr
        if edge_split:
            @pl.when(s < NS - 1)
            def _():
                pltpu.sync_copy(acc, out_hbm.at[c, pl.ds(lo, rr)])

            @pl.when(s == NS - 1)
            def _():
                pltpu.sync_copy(acc.at[pl.ds(0, tail_r)],
                                out_hbm.at[c, pl.ds(lo, tail_r)])
        else:
            @pl.when(s < NS - 1)
            def _():
                pltpu.sync_copy(
                    acc, out_hbm.at[pl.ds(lo, rr), pl.ds(c * F, F)])

            @pl.when(s == NS - 1)
            def _():
                pltpu.sync_copy(
                    acc.at[pl.ds(0, tail_r)],
                    out_hbm.at[pl.ds(lo, tail_r), pl.ds(c * F, F)])

    return agg_kernel


def _make_deg(e_pad):
    nblk = e_pad // SB // 2
    rr = 632

    @functools.partial(
        pl.kernel,
        out_type=jax.ShapeDtypeStruct((2, N, L), jnp.float32),
        mesh=_mesh(),
        compiler_params=pltpu.CompilerParams(needs_layout_passes=False),
        scratch_types=[
            pltpu.VMEM((2 * SB,), jnp.int32),
            pltpu.VMEM((QCAP,), jnp.int32),
            pltpu.VMEM((rr, L), jnp.float32),
            pltpu.SemaphoreType.DMA,
            pltpu.SemaphoreType.DMA,
        ],
    )
    def deg_kernel(dst_hbm, out_hbm, dstb, qpk, acc, is0, is1):
        c = lax.axis_index("c")
        s = lax.axis_index("s")
        lo = s * rr
        kb = c * nblk
        one = jnp.ones((L,), jnp.float32)

        _zero_rows(acc, rr, L)
        _zero_queue(qpk)

        def issue_block(k, p, sem):
            pltpu.async_copy(dst_hbm.at[pl.ds((kb + k) * SB, SB)],
                             dstb.at[pl.ds(p * SB, SB)], sem)

        def wait_block(p, sem):
            pltpu.make_async_copy(dst_hbm.at[pl.ds(0, SB)],
                                  dstb.at[pl.ds(p * SB, SB)], sem).wait()

        def drain(qn):

            def add16(i, carry):
                e = i * L
                tm = qpk[pl.ds(e, L)]
                for u in range(L):
                    r = tm[u]
                    acc[r, pl.ds(0, L)] = acc[r, pl.ds(0, L)] + one
                return carry

            lax.fori_loop(0, EB // L, add16, 0)
            _shift_queue(qpk)
            return qn - EB

        issue_block(0, 0, is0)

        def body(k, qn):
            p = lax.rem(k, 2)

            @pl.when(p == 0)
            def _():
                wait_block(0, is0)

            @pl.when(p == 1)
            def _():
                wait_block(1, is1)

            @pl.when((p == 0) & (k + 1 < nblk))
            def _():
                issue_block(k + 1, 1, is1)

            @pl.when((p == 1) & (k + 1 < nblk))
            def _():
                issue_block(k + 1, 0, is0)

            return _scan_block(None, dstb, qpk, p * SB, lo, rr, qn,
                               drain, False)

        qn = lax.fori_loop(0, nblk, body, 0)

        def add_tail(i, carry):
            r = qpk[pl.ds(i, L)][0]
            acc[r, pl.ds(0, L)] = acc[r, pl.ds(0, L)] + one
            return carry

        lax.fori_loop(0, qn, add_tail, 0)

        @pl.when(s < NS - 1)
        def _():
            pltpu.sync_copy(acc, out_hbm.at[c, pl.ds(lo, rr)])

        @pl.when(s == NS - 1)
        def _():
            pltpu.sync_copy(acc.at[pl.ds(0, N - (NS - 1) * rr)],
                            out_hbm.at[c, pl.ds(lo, N - (NS - 1) * rr)])

    return deg_kernel


def _dinv(dega_blk, degb_blk):
    return lax.rsqrt(dega_blk[:, 0:1] + degb_blk[:, 0:1] + 1.0)


def _tc1_body(x_ref, wl_ref, bl_ref, w1_ref, dega_ref, degb_ref, g1_ref):
    h0 = jnp.dot(x_ref[...], wl_ref[...],
                 preferred_element_type=jnp.float32) + bl_ref[...]
    hh = jnp.dot(h0, w1_ref[...], preferred_element_type=jnp.float32)
    g1 = _dinv(dega_ref[...], degb_ref[...]) * hh
    half = g1.shape[-1] // 2
    g1_ref[0] = g1[:, :half]
    g1_ref[1] = g1[:, half:]


def _tc2_body(s1_ref, g1a_ref, g1b_ref, dega_ref, degb_ref, b1_ref, w2_ref,
              g2_ref):
    dinv = _dinv(dega_ref[...], degb_ref[...])
    g1 = jnp.concatenate([g1a_ref[0], g1b_ref[0]], axis=-1)
    h1 = jnp.maximum(dinv * (s1_ref[...] + g1) + b1_ref[...], 0.0)
    hh2 = jnp.dot(h1, w2_ref[...], preferred_element_type=jnp.float32)
    g2_ref[...] = dinv * hh2


def _tc3_body(s2a_ref, s2b_ref, g2_ref, dega_ref, degb_ref, b2_ref, out_ref):
    dinv = _dinv(dega_ref[...], degb_ref[...])
    out_ref[...] = dinv * (s2a_ref[...] + s2b_ref[...] + g2_ref[...]) \
        + b2_ref[...]


def _row_spec(cols):
    return pl.BlockSpec((BR, cols), lambda i: (i, 0))


def _full_spec(shape):
    return pl.BlockSpec(shape, lambda i: (0,) * len(shape))


def kernel(edge_index, x, Wl, bl, W1, b1, W2, b2):
    nfeat = x.shape[1]
    h2 = Wl.shape[1]
    nhid = W2.shape[1]
    e = edge_index.shape[1]

    src = edge_index[0].astype(jnp.int32)
    dst = edge_index[1].astype(jnp.int32)
    e_pad = pl.cdiv(e, 2 * SB) * (2 * SB)
    if e_pad != e:
        pad = e_pad - e
        src = jnp.concatenate([src, jnp.zeros((pad,), jnp.int32)])
        dst = jnp.concatenate([dst, jnp.full((pad,), N, jnp.int32)])

    deg16 = _make_deg(e_pad)(dst)
    dega, degb = deg16[0], deg16[1]

    grid = (pl.cdiv(N, BR),)
    tc1 = pl.pallas_call(
        _tc1_body,
        grid=grid,
        in_specs=[_row_spec(nfeat), _full_spec((nfeat, h2)),
                  _full_spec((1, h2)), _full_spec((h2, h2)), _row_spec(L),
                  _row_spec(L)],
        out_specs=pl.BlockSpec((2, BR, h2 // 2), lambda i: (0, i, 0)),
        out_shape=jax.ShapeDtypeStruct((2, N, h2 // 2), jnp.float32),
    )
    g1 = tc1(x, Wl, bl.reshape(1, h2), W1, dega, degb)

    s1 = _make_agg(e_pad, h2 // 2, "feat")(g1.reshape(2 * N, h2 // 2),
                                           src, dst)

    tc2 = pl.pallas_call(
        _tc2_body,
        grid=grid,
        in_specs=[_row_spec(h2),
                  pl.BlockSpec((1, BR, h2 // 2), lambda i: (0, i, 0)),
                  pl.BlockSpec((1, BR, h2 // 2), lambda i: (1, i, 0)),
                  _row_spec(L), _row_spec(L),
                  _full_spec((1, h2)), _full_spec((h2, nhid))],
        out_specs=_row_spec(nhid),
        out_shape=jax.ShapeDtypeStruct((N, nhid), jnp.float32),
    )
    g2 = tc2(s1, g1, g1, dega, degb, b1.reshape(1, h2), W2)

    s2 = _make_agg(e_pad, nhid, "edge")(g2, src, dst)

    tc3 = pl.pallas_call(
        _tc3_body,
        grid=grid,
        in_specs=[_row_spec(nhid), _row_spec(nhid), _row_spec(nhid),
                  _row_spec(L), _row_spec(L), _full_spec((1, nhid))],
        out_specs=_row_spec(nhid),
        out_shape=jax.ShapeDtypeStruct((N, nhid), jnp.float32),
    )
    return tc3(s2[0], s2[1], g2, dega, degb, b2.reshape(1, nhid))


# speedup vs baseline: 1.1389x; 1.1389x over previous
import functools

import jax
import jax.numpy as jnp
from jax import lax
from jax.experimental import pallas as pl
from jax.experimental.pallas import tpu as pltpu
from jax.experimental.pallas import tpu_sc as plsc

N = 10000
NC = 2
NS = 16
NW = NC * NS
L = 16
R = 320
EB = 128
SB = 512
QCAP = 672
BR = 256

_mesh = functools.partial(
    plsc.VectorSubcoreMesh, core_axis_name="c", subcore_axis_name="s")


def _zero_rows(ref, nrows, ncols):
    z = jnp.zeros((L,), jnp.float32)
    fw = ncols // L

    def body(t, carry):
        i = t // fw
        j = t - i * fw
        ref[i, pl.ds(j * L, L)] = z
        return carry

    lax.fori_loop(0, nrows * fw, body, 0)


def _zero_queue(qref):
    z = jnp.zeros((L,), jnp.int32)

    def body(i, carry):
        qref[pl.ds(i * L, L)] = z
        return carry

    lax.fori_loop(0, QCAP // L, body, 0)


def _scan_block(srcb, dstb, qpk, base, lo, rr, qn, drain, with_src):
    offs = []
    for ch in range(SB // L):
        off = base + ch * L
        d16 = dstb[pl.ds(off, L)]
        ok = (d16 - lo).astype(jnp.uint32) < rr
        offs.append(qn)
        qn = qn + plsc.all_reduce_population_count(ok)[0]
    for ch in range(SB // L):
        off = base + ch * L
        d16 = dstb[pl.ds(off, L)]
        loc = d16 - lo
        ok = (loc >= 0) & (loc < rr)
        if with_src:
            s16 = srcb[pl.ds(off, L)]
            val = s16 * 1024 + loc
        else:
            val = loc
        key = jnp.where(ok, 0, 1)
        _, vs = plsc.sort_key_val(key, val)
        qpk[pl.ds(offs[ch], L)] = vs
    return lax.while_loop(lambda q: q >= EB, drain, qn)


def _shift_queue(qref):
    for k in range((QCAP - EB) // L):
        t = qref[pl.ds(EB + k * L, L)]
        qref[pl.ds(k * L, L)] = t


def _make_agg(e_pad, F, mode):
    edge_split = mode == "edge"
    nblk = e_pad // SB // 2 if edge_split else e_pad // SB
    rr = 632
    fw = F // L
    out_shape = (2, N, F) if edge_split else (N, 2 * F)

    @functools.partial(
        pl.kernel,
        out_type=jax.ShapeDtypeStruct(out_shape, jnp.float32),
        mesh=_mesh(),
        compiler_params=pltpu.CompilerParams(needs_layout_passes=False),
        scratch_types=[
            pltpu.VMEM((2 * SB,), jnp.int32),
            pltpu.VMEM((2 * SB,), jnp.int32),
            pltpu.VMEM((QCAP,), jnp.int32),
            pltpu.VMEM((EB,), jnp.int32),
            pltpu.VMEM((EB, F), jnp.float32),
            pltpu.VMEM((rr, F), jnp.float32),
            pltpu.SemaphoreType.DMA,
            pltpu.SemaphoreType.DMA,
            pltpu.SemaphoreType.DMA,
        ],
    )
    def agg_kernel(g_hbm, src_hbm, dst_hbm, out_hbm,
                   srcb, dstb, qpk, qsrc, rows, acc, gsem, is0, is1):
        c = lax.axis_index("c")
        s = lax.axis_index("s")
        lo = s * rr
        kb = c * nblk if edge_split else 0
        gbase = 0 if edge_split else c * N

        _zero_rows(acc, rr, F)
        _zero_queue(qpk)

        def issue_block(k, p, sem):
            pltpu.async_copy(src_hbm.at[pl.ds((kb + k) * SB, SB)],
                             srcb.at[pl.ds(p * SB, SB)], sem)
            pltpu.async_copy(dst_hbm.at[pl.ds((kb + k) * SB, SB)],
                             dstb.at[pl.ds(p * SB, SB)], sem)

        def wait_block(p, sem):
            pltpu.make_async_copy(src_hbm.at[pl.ds(0, SB)],
                                  srcb.at[pl.ds(p * SB, SB)], sem).wait()
            pltpu.make_async_copy(dst_hbm.at[pl.ds(0, SB)],
                                  dstb.at[pl.ds(p * SB, SB)], sem).wait()

        def do_add(i, r):
            for j in range(fw):
                sl = pl.ds(j * L, L)
                acc[r, sl] = acc[r, sl] + rows[i, sl]

        def drain(qn):
            for k in range(EB // L):
                t = qpk[pl.ds(k * L, L)]
                qsrc[pl.ds(k * L, L)] = jnp.minimum(
                    lax.shift_right_logical(t, 10), N - 1) + gbase
            pltpu.async_copy(g_hbm.at[qsrc], rows, gsem).wait()

            def add16(i, carry):
                e = i * L
                tm = jnp.bitwise_and(qpk[pl.ds(e, L)], 1023)
                for u in range(L):
                    r = tm[u]
                    for j in range(fw):
                        sl = pl.ds(j * L, L)
                        acc[r, sl] = acc[r, sl] + rows[e + u, sl]
                return carry

            lax.fori_loop(0, EB // L, add16, 0)
            _shift_queue(qpk)
            return qn - EB

        issue_block(0, 0, is0)

        def body(k, qn):
            p = lax.rem(k, 2)

            @pl.when(p == 0)
            def _():
                wait_block(0, is0)

            @pl.when(p == 1)
            def _():
                wait_block(1, is1)

            @pl.when((p == 0) & (k + 1 < nblk))
            def _():
                issue_block(k + 1, 1, is1)

            @pl.when((p == 1) & (k + 1 < nblk))
            def _():
                issue_block(k + 1, 0, is0)

            return _scan_block(srcb, dstb, qpk, p * SB, lo, rr, qn,
                               drain, True)

        qn = lax.fori_loop(0, nblk, body, 0)

        for k in range(EB // L):
            t = qpk[pl.ds(k * L, L)]
            qsrc[pl.ds(k * L, L)] = jnp.minimum(
                lax.shift_right_logical(t, 10), N - 1) + gbase
        pltpu.async_copy(g_hbm.at[qsrc], rows, gsem).wait()

        def add_tail(i, carry):
            do_add(i, jnp.bitwise_and(qpk[pl.ds(i, L)][0], 1023))
            return carry

        lax.fori_loop(0, qn, add_tail, 0)

        tail_r = N - (NS - 1) * rr
        if edge_split:
            @pl.when(s < NS - 1)
            def _():
                pltpu.sync_copy(acc, out_hbm.at[c, pl.ds(lo, rr)])

            @pl.when(s == NS - 1)
            def _():
                pltpu.sync_copy(acc.at[pl.ds(0, tail_r)],
                                out_hbm.at[c, pl.ds(lo, tail_r)])
        else:
            @pl.when(s < NS - 1)
            def _():
                pltpu.sync_copy(
                    acc, out_hbm.at[pl.ds(lo, rr), pl.ds(c * F, F)])

            @pl.when(s == NS - 1)
            def _():
                pltpu.sync_copy(
                    acc.at[pl.ds(0, tail_r)],
                    out_hbm.at[pl.ds(lo, tail_r), pl.ds(c * F, F)])

    return agg_kernel


def _make_deg(e_pad):
    nblk = e_pad // SB // 2
    rr = 632

    @functools.partial(
        pl.kernel,
        out_type=jax.ShapeDtypeStruct((2, N, L), jnp.float32),
        mesh=_mesh(),
        compiler_params=pltpu.CompilerParams(needs_layout_passes=False),
        scratch_types=[
            pltpu.VMEM((2 * SB,), jnp.int32),
            pltpu.VMEM((QCAP,), jnp.int32),
            pltpu.VMEM((rr, L), jnp.float32),
            pltpu.SemaphoreType.DMA,
            pltpu.SemaphoreType.DMA,
        ],
    )
    def deg_kernel(dst_hbm, out_hbm, dstb, qpk, acc, is0, is1):
        c = lax.axis_index("c")
        s = lax.axis_index("s")
        lo = s * rr
        kb = c * nblk
        one = jnp.ones((L,), jnp.float32)

        _zero_rows(acc, rr, L)
        _zero_queue(qpk)

        def issue_block(k, p, sem):
            pltpu.async_copy(dst_hbm.at[pl.ds((kb + k) * SB, SB)],
                             dstb.at[pl.ds(p * SB, SB)], sem)

        def wait_block(p, sem):
            pltpu.make_async_copy(dst_hbm.at[pl.ds(0, SB)],
                                  dstb.at[pl.ds(p * SB, SB)], sem).wait()

        def drain(qn):

            def add16(i, carry):
                e = i * L
                tm = qpk[pl.ds(e, L)]
                for u in range(L):
                    r = tm[u]
                    acc[r, pl.ds(0, L)] = acc[r, pl.ds(0, L)] + one
                return carry

            lax.fori_loop(0, EB // L, add16, 0)
            _shift_queue(qpk)
            return qn - EB

        issue_block(0, 0, is0)

        def body(k, qn):
            p = lax.rem(k, 2)

            @pl.when(p == 0)
            def _():
                wait_block(0, is0)

            @pl.when(p == 1)
            def _():
                wait_block(1, is1)

            @pl.when((p == 0) & (k + 1 < nblk))
            def _():
                issue_block(k + 1, 1, is1)

            @pl.when((p == 1) & (k + 1 < nblk))
            def _():
                issue_block(k + 1, 0, is0)

            return _scan_block(None, dstb, qpk, p * SB, lo, rr, qn,
                               drain, False)

        qn = lax.fori_loop(0, nblk, body, 0)

        def add_tail(i, carry):
            r = qpk[pl.ds(i, L)][0]
            acc[r, pl.ds(0, L)] = acc[r, pl.ds(0, L)] + one
            return carry

        lax.fori_loop(0, qn, add_tail, 0)

        @pl.when(s < NS - 1)
        def _():
            pltpu.sync_copy(acc, out_hbm.at[c, pl.ds(lo, rr)])

        @pl.when(s == NS - 1)
        def _():
            pltpu.sync_copy(acc.at[pl.ds(0, N - (NS - 1) * rr)],
                            out_hbm.at[c, pl.ds(lo, N - (NS - 1) * rr)])

    return deg_kernel


def _dinv(dega_blk, degb_blk):
    return lax.rsqrt(dega_blk[:, 0:1] + degb_blk[:, 0:1] + 1.0)


def _tc1_body(x_ref, wl_ref, bl_ref, w1_ref, dega_ref, degb_ref, g1_ref):
    h0 = jnp.dot(x_ref[...], wl_ref[...],
                 preferred_element_type=jnp.float32) + bl_ref[...]
    hh = jnp.dot(h0, w1_ref[...], preferred_element_type=jnp.float32)
    g1 = _dinv(dega_ref[...], degb_ref[...]) * hh
    half = g1.shape[-1] // 2
    g1_ref[0] = g1[:, :half]
    g1_ref[1] = g1[:, half:]


def _tc2_body(s1_ref, g1a_ref, g1b_ref, dega_ref, degb_ref, b1_ref, w2_ref,
              g2_ref):
    dinv = _dinv(dega_ref[...], degb_ref[...])
    g1 = jnp.concatenate([g1a_ref[0], g1b_ref[0]], axis=-1)
    h1 = jnp.maximum(dinv * (s1_ref[...] + g1) + b1_ref[...], 0.0)
    hh2 = jnp.dot(h1, w2_ref[...], preferred_element_type=jnp.float32)
    g2_ref[...] = dinv * hh2


def _tc3_body(s2a_ref, s2b_ref, g2_ref, dega_ref, degb_ref, b2_ref, out_ref):
    dinv = _dinv(dega_ref[...], degb_ref[...])
    out_ref[...] = dinv * (s2a_ref[...] + s2b_ref[...] + g2_ref[...]) \
        + b2_ref[...]


def _row_spec(cols):
    return pl.BlockSpec((BR, cols), lambda i: (i, 0))


def _full_spec(shape):
    return pl.BlockSpec(shape, lambda i: (0,) * len(shape))


def kernel(edge_index, x, Wl, bl, W1, b1, W2, b2):
    nfeat = x.shape[1]
    h2 = Wl.shape[1]
    nhid = W2.shape[1]
    e = edge_index.shape[1]

    src = edge_index[0].astype(jnp.int32)
    dst = edge_index[1].astype(jnp.int32)
    e_pad = pl.cdiv(e, 2 * SB) * (2 * SB)
    if e_pad != e:
        pad = e_pad - e
        src = jnp.concatenate([src, jnp.zeros((pad,), jnp.int32)])
        dst = jnp.concatenate([dst, jnp.full((pad,), N, jnp.int32)])

    deg16 = _make_deg(e_pad)(dst)
    dega, degb = deg16[0], deg16[1]

    grid = (pl.cdiv(N, BR),)
    tc1 = pl.pallas_call(
        _tc1_body,
        grid=grid,
        in_specs=[_row_spec(nfeat), _full_spec((nfeat, h2)),
                  _full_spec((1, h2)), _full_spec((h2, h2)), _row_spec(L),
                  _row_spec(L)],
        out_specs=pl.BlockSpec((2, BR, h2 // 2), lambda i: (0, i, 0)),
        out_shape=jax.ShapeDtypeStruct((2, N, h2 // 2), jnp.float32),
    )
    g1 = tc1(x, Wl, bl.reshape(1, h2), W1, dega, degb)

    s1 = _make_agg(e_pad, h2 // 2, "feat")(g1.reshape(2 * N, h2 // 2),
                                           src, dst)

    tc2 = pl.pallas_call(
        _tc2_body,
        grid=grid,
        in_specs=[_row_spec(h2),
                  pl.BlockSpec((1, BR, h2 // 2), lambda i: (0, i, 0)),
                  pl.BlockSpec((1, BR, h2 // 2), lambda i: (1, i, 0)),
                  _row_spec(L), _row_spec(L),
                  _full_spec((1, h2)), _full_spec((h2, nhid))],
        out_specs=_row_spec(nhid),
        out_shape=jax.ShapeDtypeStruct((N, nhid), jnp.float32),
    )
    g2 = tc2(s1, g1, g1, dega, degb, b1.reshape(1, h2), W2)

    s2 = _make_agg(e_pad, nhid, "edge")(g2, src, dst)

    tc3 = pl.pallas_call(
        _tc3_body,
        grid=grid,
        in_specs=[_row_spec(nhid), _row_spec(nhid), _row_spec(nhid),
                  _row_spec(L), _row_spec(L), _full_spec((1, nhid))],
        out_specs=_row_spec(nhid),
        out_shape=jax.ShapeDtypeStruct((N, nhid), jnp.float32),
    )
    return tc3(s2[0], s2[1], g2, dega, degb, b2.reshape(1, nhid))
